# baseline (speedup 1.0000x reference)
import jax
import jax.numpy as jnp
from jax import lax
from jax.experimental import pallas as pl
from jax.experimental.pallas import tpu as pltpu
from jax.experimental.pallas import tpu_sc as plsc

_NUM_EXPERTS = 8
_TOP_K = 2
_HIDDEN = 1024
_INTER = 2816
_NUM_TOKENS = 32

_BI = 1408
_GRP = 16


def _sc_routing(router_logits):
    mesh = plsc.VectorSubcoreMesh(core_axis_name="c", subcore_axis_name="s")
    ngroups = _NUM_TOKENS // _GRP

    @pl.kernel(
        out_type=jax.ShapeDtypeStruct((_NUM_TOKENS * _NUM_EXPERTS,),
                                      jnp.float32),
        mesh=mesh,
        scratch_types=[
            pltpu.VMEM((_NUM_TOKENS, _NUM_EXPERTS), jnp.float32),
            pltpu.VMEM((_GRP * _NUM_EXPERTS,), jnp.float32),
        ],
        compiler_params=pltpu.CompilerParams(needs_layout_passes=False),
    )
    def routing_kernel(rl_hbm, out_hbm, lbuf, obuf):
        g = lax.axis_index("s")

        @pl.when((lax.axis_index("c") == 0) & (g < ngroups))
        def _():
            pltpu.sync_copy(rl_hbm, lbuf)
            lane = lax.iota(jnp.int32, _GRP)
            tidx = g * _GRP + lane
            ls = [
                plsc.load_gather(lbuf, [tidx, jnp.full((_GRP,), e,
                                                       jnp.int32)])
                for e in range(_NUM_EXPERTS)
            ]
            m1 = ls[0]
            for e in range(1, _NUM_EXPERTS):
                m1 = jnp.maximum(m1, ls[e])
            big = jnp.full((_GRP,), _NUM_EXPERTS, jnp.int32)
            i1 = big
            for e in range(_NUM_EXPERTS):
                cand = jnp.where(ls[e] == m1,
                                 jnp.full((_GRP,), e, jnp.int32), big)
                i1 = jnp.minimum(i1, cand)
            neg = jnp.full((_GRP,), -1e30, jnp.float32)
            ls2 = [jnp.where(i1 == e, neg, ls[e])
                   for e in range(_NUM_EXPERTS)]
            m2 = ls2[0]
            for e in range(1, _NUM_EXPERTS):
                m2 = jnp.maximum(m2, ls2[e])
            i2 = big
            for e in range(_NUM_EXPERTS):
                cand = jnp.where(ls2[e] == m2,
                                 jnp.full((_GRP,), e, jnp.int32), big)
                i2 = jnp.minimum(i2, cand)
            w1 = 1.0 / (1.0 + jnp.exp(m2 - m1))
            w2 = 1.0 - w1
            for k in range(_NUM_EXPERTS):
                obuf[pl.ds(k * _GRP, _GRP)] = jnp.zeros((_GRP,),
                                                        jnp.float32)
            plsc.store_scatter(obuf, [lane * _NUM_EXPERTS + i1], w1)
            plsc.store_scatter(obuf, [lane * _NUM_EXPERTS + i2], w2)
            pltpu.sync_copy(
                obuf, out_hbm.at[pl.ds(g * _GRP * _NUM_EXPERTS,
                                       _GRP * _NUM_EXPERTS)])

    return routing_kernel(router_logits).reshape(_NUM_TOKENS, _NUM_EXPERTS)


def _moe_body(x_ref, wte_ref, w13_ref, w2_ref, out_ref):
    e = pl.program_id(0)
    i = pl.program_id(1)

    @pl.when((e == 0) & (i == 0))
    def _():
        out_ref[...] = jnp.zeros_like(out_ref)

    xb = x_ref[...].astype(jnp.bfloat16)
    gate_w = w13_ref[0, 0].astype(jnp.bfloat16)
    up_w = w13_ref[0, 1].astype(jnp.bfloat16)
    dims = (((1,), (1,)), ((), ()))
    gate = jax.lax.dot_general(xb, gate_w, dims,
                               preferred_element_type=jnp.float32)
    up = jax.lax.dot_general(xb, up_w, dims,
                             preferred_element_type=jnp.float32)
    act = gate * jax.nn.sigmoid(gate) * up

    eidx = jax.lax.broadcasted_iota(jnp.int32, (_NUM_TOKENS, _NUM_EXPERTS), 1)
    scale = jnp.sum(jnp.where(eidx == e, wte_ref[...], 0.0), axis=-1,
                    keepdims=True)
    actb = (act * scale).astype(jnp.bfloat16)
    w2b = w2_ref[0].astype(jnp.bfloat16)
    out_ref[...] += jax.lax.dot_general(
        actb, w2b, (((1,), (1,)), ((), ())),
        preferred_element_type=jnp.float32)


def kernel(x, router_logits, w13, w2):
    wte = _sc_routing(router_logits)
    w13r = w13.reshape(_NUM_EXPERTS, 2, _INTER, _HIDDEN)
    grid = (_NUM_EXPERTS, _INTER // _BI)
    return pl.pallas_call(
        _moe_body,
        grid=grid,
        in_specs=[
            pl.BlockSpec((_NUM_TOKENS, _HIDDEN), lambda e, i: (0, 0)),
            pl.BlockSpec((_NUM_TOKENS, _NUM_EXPERTS), lambda e, i: (0, 0)),
            pl.BlockSpec((1, 2, _BI, _HIDDEN), lambda e, i: (e, 0, i, 0)),
            pl.BlockSpec((1, _HIDDEN, _BI), lambda e, i: (e, 0, i)),
        ],
        out_specs=pl.BlockSpec((_NUM_TOKENS, _HIDDEN), lambda e, i: (0, 0)),
        out_shape=jax.ShapeDtypeStruct((_NUM_TOKENS, _HIDDEN), jnp.float32),
        compiler_params=pltpu.CompilerParams(
            dimension_semantics=("arbitrary", "arbitrary")),
    )(x, wte, w13r, w2)


# speedup vs baseline: 1.0018x; 1.0018x over previous
import jax
import jax.numpy as jnp
from jax import lax
from jax.experimental import pallas as pl
from jax.experimental.pallas import tpu as pltpu
from jax.experimental.pallas import tpu_sc as plsc

_NUM_EXPERTS = 8
_TOP_K = 2
_HIDDEN = 1024
_INTER = 2816
_NUM_TOKENS = 32

_BI = 1408
_GRP = 16


def _sc_routing(router_logits):
    mesh = plsc.VectorSubcoreMesh(core_axis_name="c", subcore_axis_name="s")
    ngroups = _NUM_TOKENS // _GRP

    @pl.kernel(
        out_type=jax.ShapeDtypeStruct((_NUM_TOKENS * _NUM_EXPERTS,),
                                      jnp.float32),
        mesh=mesh,
        scratch_types=[
            pltpu.VMEM((_NUM_TOKENS, _NUM_EXPERTS), jnp.float32),
            pltpu.VMEM((_GRP * _NUM_EXPERTS,), jnp.float32),
        ],
        compiler_params=pltpu.CompilerParams(needs_layout_passes=False),
    )
    def routing_kernel(rl_hbm, out_hbm, lbuf, obuf):
        g = lax.axis_index("s")

        @pl.when((lax.axis_index("c") == 0) & (g < ngroups))
        def _():
            pltpu.sync_copy(rl_hbm, lbuf)
            lane = lax.iota(jnp.int32, _GRP)
            tidx = g * _GRP + lane
            ls = [
                plsc.load_gather(lbuf, [tidx, jnp.full((_GRP,), e,
                                                       jnp.int32)])
                for e in range(_NUM_EXPERTS)
            ]
            m1 = ls[0]
            for e in range(1, _NUM_EXPERTS):
                m1 = jnp.maximum(m1, ls[e])
            big = jnp.full((_GRP,), _NUM_EXPERTS, jnp.int32)
            i1 = big
            for e in range(_NUM_EXPERTS):
                cand = jnp.where(ls[e] == m1,
                                 jnp.full((_GRP,), e, jnp.int32), big)
                i1 = jnp.minimum(i1, cand)
            neg = jnp.full((_GRP,), -1e30, jnp.float32)
            ls2 = [jnp.where(i1 == e, neg, ls[e])
                   for e in range(_NUM_EXPERTS)]
            m2 = ls2[0]
            for e in range(1, _NUM_EXPERTS):
                m2 = jnp.maximum(m2, ls2[e])
            i2 = big
            for e in range(_NUM_EXPERTS):
                cand = jnp.where(ls2[e] == m2,
                                 jnp.full((_GRP,), e, jnp.int32), big)
                i2 = jnp.minimum(i2, cand)
            w1 = 1.0 / (1.0 + jnp.exp(m2 - m1))
            w2 = 1.0 - w1
            for k in range(_NUM_EXPERTS):
                obuf[pl.ds(k * _GRP, _GRP)] = jnp.zeros((_GRP,),
                                                        jnp.float32)
            plsc.store_scatter(obuf, [lane * _NUM_EXPERTS + i1], w1)
            plsc.store_scatter(obuf, [lane * _NUM_EXPERTS + i2], w2)
            pltpu.sync_copy(
                obuf, out_hbm.at[pl.ds(g * _GRP * _NUM_EXPERTS,
                                       _GRP * _NUM_EXPERTS)])

    return routing_kernel(router_logits).reshape(_NUM_TOKENS, _NUM_EXPERTS)


def _moe_body(x_ref, w13_ref, w2_ref, acc_ref):
    i = pl.program_id(1)

    xb = x_ref[...].astype(jnp.bfloat16)
    gate_w = w13_ref[0, 0].astype(jnp.bfloat16)
    up_w = w13_ref[0, 1].astype(jnp.bfloat16)
    dims = (((1,), (1,)), ((), ()))
    gate = jax.lax.dot_general(xb, gate_w, dims,
                               preferred_element_type=jnp.float32)
    up = jax.lax.dot_general(xb, up_w, dims,
                             preferred_element_type=jnp.float32)
    act = gate * jax.nn.sigmoid(gate) * up
    w2b = w2_ref[0].astype(jnp.bfloat16)
    partial = jax.lax.dot_general(
        act.astype(jnp.bfloat16), w2b, (((1,), (1,)), ((), ())),
        preferred_element_type=jnp.float32)

    @pl.when(i == 0)
    def _():
        acc_ref[0] = partial

    @pl.when(i != 0)
    def _():
        acc_ref[0] += partial


def _combine_body(acc_ref, wte_ref, out_ref):
    total = jnp.zeros((_NUM_TOKENS, _HIDDEN), jnp.float32)
    for e in range(_NUM_EXPERTS):
        total = total + acc_ref[e] * wte_ref[:, e:e + 1]
    out_ref[...] = total


def kernel(x, router_logits, w13, w2):
    wte = _sc_routing(router_logits)
    w13r = w13.reshape(_NUM_EXPERTS, 2, _INTER, _HIDDEN)
    grid = (_NUM_EXPERTS, _INTER // _BI)
    acc = pl.pallas_call(
        _moe_body,
        grid=grid,
        in_specs=[
            pl.BlockSpec((_NUM_TOKENS, _HIDDEN), lambda e, i: (0, 0)),
            pl.BlockSpec((1, 2, _BI, _HIDDEN), lambda e, i: (e, 0, i, 0)),
            pl.BlockSpec((1, _HIDDEN, _BI), lambda e, i: (e, 0, i)),
        ],
        out_specs=pl.BlockSpec((1, _NUM_TOKENS, _HIDDEN),
                               lambda e, i: (e, 0, 0)),
        out_shape=jax.ShapeDtypeStruct(
            (_NUM_EXPERTS, _NUM_TOKENS, _HIDDEN), jnp.float32),
        compiler_params=pltpu.CompilerParams(
            dimension_semantics=("arbitrary", "arbitrary")),
    )(x, w13r, w2)
    return pl.pallas_call(
        _combine_body,
        out_shape=jax.ShapeDtypeStruct((_NUM_TOKENS, _HIDDEN), jnp.float32),
    )(acc, wte)


# speedup vs baseline: 1.0323x; 1.0305x over previous
import jax
import jax.numpy as jnp
from jax import lax
from jax.experimental import pallas as pl
from jax.experimental.pallas import tpu as pltpu
from jax.experimental.pallas import tpu_sc as plsc

_NUM_EXPERTS = 8
_TOP_K = 2
_HIDDEN = 1024
_INTER = 2816
_NUM_TOKENS = 32

_BI = 1408
_GRP = 16


def _sc_routing(router_logits):
    mesh = plsc.VectorSubcoreMesh(core_axis_name="c", subcore_axis_name="s",
                                  num_cores=1)
    ngroups = _NUM_TOKENS // _GRP

    @pl.kernel(
        out_type=jax.ShapeDtypeStruct((_NUM_TOKENS * _NUM_EXPERTS,),
                                      jnp.float32),
        mesh=mesh,
        scratch_types=[
            pltpu.VMEM((_NUM_TOKENS, _NUM_EXPERTS), jnp.float32),
            pltpu.VMEM((_GRP * _NUM_EXPERTS,), jnp.float32),
        ],
        compiler_params=pltpu.CompilerParams(needs_layout_passes=False),
    )
    def routing_kernel(rl_hbm, out_hbm, lbuf, obuf):
        g = lax.axis_index("s")

        @pl.when((lax.axis_index("c") == 0) & (g < ngroups))
        def _():
            pltpu.sync_copy(rl_hbm, lbuf)
            lane = lax.iota(jnp.int32, _GRP)
            tidx = g * _GRP + lane
            ls = [
                plsc.load_gather(lbuf, [tidx, jnp.full((_GRP,), e,
                                                       jnp.int32)])
                for e in range(_NUM_EXPERTS)
            ]
            m1 = ls[0]
            for e in range(1, _NUM_EXPERTS):
                m1 = jnp.maximum(m1, ls[e])
            big = jnp.full((_GRP,), _NUM_EXPERTS, jnp.int32)
            i1 = big
            for e in range(_NUM_EXPERTS):
                cand = jnp.where(ls[e] == m1,
                                 jnp.full((_GRP,), e, jnp.int32), big)
                i1 = jnp.minimum(i1, cand)
            neg = jnp.full((_GRP,), -1e30, jnp.float32)
            ls2 = [jnp.where(i1 == e, neg, ls[e])
                   for e in range(_NUM_EXPERTS)]
            m2 = ls2[0]
            for e in range(1, _NUM_EXPERTS):
                m2 = jnp.maximum(m2, ls2[e])
            i2 = big
            for e in range(_NUM_EXPERTS):
                cand = jnp.where(ls2[e] == m2,
                                 jnp.full((_GRP,), e, jnp.int32), big)
                i2 = jnp.minimum(i2, cand)
            w1 = 1.0 / (1.0 + jnp.exp(m2 - m1))
            w2 = 1.0 - w1
            for k in range(_NUM_EXPERTS):
                obuf[pl.ds(k * _GRP, _GRP)] = jnp.zeros((_GRP,),
                                                        jnp.float32)
            plsc.store_scatter(obuf, [lane * _NUM_EXPERTS + i1], w1)
            plsc.store_scatter(obuf, [lane * _NUM_EXPERTS + i2], w2)
            pltpu.sync_copy(
                obuf, out_hbm.at[pl.ds(g * _GRP * _NUM_EXPERTS,
                                       _GRP * _NUM_EXPERTS)])

    return routing_kernel(router_logits).reshape(_NUM_TOKENS, _NUM_EXPERTS)


def _moe_body(x_ref, w13_ref, w2_ref, acc_ref):
    i = pl.program_id(1)

    xb = x_ref[...].astype(jnp.bfloat16)
    gate_w = w13_ref[0, 0].astype(jnp.bfloat16)
    up_w = w13_ref[0, 1].astype(jnp.bfloat16)
    dims = (((1,), (1,)), ((), ()))
    gate = jax.lax.dot_general(xb, gate_w, dims,
                               preferred_element_type=jnp.float32)
    up = jax.lax.dot_general(xb, up_w, dims,
                             preferred_element_type=jnp.float32)
    act = gate * jax.nn.sigmoid(gate) * up
    w2b = w2_ref[0].astype(jnp.bfloat16)
    partial = jax.lax.dot_general(
        act.astype(jnp.bfloat16), w2b, (((1,), (1,)), ((), ())),
        preferred_element_type=jnp.float32)

    @pl.when(i == 0)
    def _():
        acc_ref[0] = partial

    @pl.when(i != 0)
    def _():
        acc_ref[0] += partial


def _combine_body(acc_ref, wte_ref, out_ref):
    total = jnp.zeros((_NUM_TOKENS, _HIDDEN), jnp.float32)
    for e in range(_NUM_EXPERTS):
        total = total + acc_ref[e] * wte_ref[:, e:e + 1]
    out_ref[...] = total


def kernel(x, router_logits, w13, w2):
    wte = _sc_routing(router_logits)
    w13r = w13.reshape(_NUM_EXPERTS, 2, _INTER, _HIDDEN)
    grid = (_NUM_EXPERTS, _INTER // _BI)
    acc = pl.pallas_call(
        _moe_body,
        grid=grid,
        in_specs=[
            pl.BlockSpec((_NUM_TOKENS, _HIDDEN), lambda e, i: (0, 0)),
            pl.BlockSpec((1, 2, _BI, _HIDDEN), lambda e, i: (e, 0, i, 0)),
            pl.BlockSpec((1, _HIDDEN, _BI), lambda e, i: (e, 0, i)),
        ],
        out_specs=pl.BlockSpec((1, _NUM_TOKENS, _HIDDEN),
                               lambda e, i: (e, 0, 0)),
        out_shape=jax.ShapeDtypeStruct(
            (_NUM_EXPERTS, _NUM_TOKENS, _HIDDEN), jnp.float32),
        compiler_params=pltpu.CompilerParams(
            dimension_semantics=("arbitrary", "arbitrary")),
    )(x, w13r, w2)
    return pl.pallas_call(
        _combine_body,
        out_shape=jax.ShapeDtypeStruct((_NUM_TOKENS, _HIDDEN), jnp.float32),
    )(acc, wte)
